# baseline (speedup 1.0000x reference)
import numpy as np

import jax
import jax.numpy as jnp
from jax.experimental import pallas as pl
from jax.experimental.pallas import tpu as pltpu
from jax.sharding import Mesh, PartitionSpec as P

_N = 10000
_ROWS = 200


def _layer1_body(feat_ref, adj_ref, w1_ref, b1_ref, x1_ref, u_ref):
    @pl.when(pl.program_id(0) == 0)
    def _():
        u = jnp.dot(feat_ref[...], w1_ref[...],
                    preferred_element_type=jnp.float32)
        u_ref[...] = u.astype(jnp.bfloat16)

    a = adj_ref[...].astype(jnp.bfloat16)
    h = jnp.dot(a, u_ref[...], preferred_element_type=jnp.float32)
    x1_ref[...] = jnp.maximum(h + b1_ref[...], 0.0)


def _layer2_body(x1_ref, adj_ref, w2_ref, b2_ref, out_ref, v_ref):
    @pl.when(pl.program_id(0) == 0)
    def _():
        v = jnp.dot(x1_ref[...], w2_ref[...],
                    preferred_element_type=jnp.float32)
        v_ref[...] = v.astype(jnp.bfloat16)

    a = adj_ref[...].astype(jnp.bfloat16)
    h = jnp.dot(a, v_ref[...], preferred_element_type=jnp.float32)
    h = h + b2_ref[...]
    m = jnp.max(h, axis=1, keepdims=True)
    e = jnp.exp(h - m)
    s = jnp.sum(e, axis=1, keepdims=True)
    out_ref[...] = h - m - jnp.log(s)


def _two_layer_local(feature, adj_local, W1, b1r, W2, b2r):
    n_loc = adj_local.shape[0]
    f_in = feature.shape[1]
    hid = W1.shape[1]
    dim = W2.shape[1]
    nsteps = n_loc // _ROWS

    x1_local = pl.pallas_call(
        _layer1_body,
        grid=(nsteps,),
        in_specs=[
            pl.BlockSpec((_N, f_in), lambda i: (0, 0)),
            pl.BlockSpec((_ROWS, _N), lambda i: (i, 0)),
            pl.BlockSpec((f_in, hid), lambda i: (0, 0)),
            pl.BlockSpec((1, hid), lambda i: (0, 0)),
        ],
        out_specs=pl.BlockSpec((_ROWS, hid), lambda i: (i, 0)),
        out_shape=jax.ShapeDtypeStruct((n_loc, hid), jnp.float32),
        scratch_shapes=[pltpu.VMEM((_N, hid), jnp.bfloat16)],
    )(feature, adj_local, W1, b1r)

    x1_full = jax.lax.all_gather(x1_local, "x", axis=0, tiled=True)

    out_local = pl.pallas_call(
        _layer2_body,
        grid=(nsteps,),
        in_specs=[
            pl.BlockSpec((_N, hid), lambda i: (0, 0)),
            pl.BlockSpec((_ROWS, _N), lambda i: (i, 0)),
            pl.BlockSpec((hid, dim), lambda i: (0, 0)),
            pl.BlockSpec((1, dim), lambda i: (0, 0)),
        ],
        out_specs=pl.BlockSpec((_ROWS, dim), lambda i: (i, 0)),
        out_shape=jax.ShapeDtypeStruct((n_loc, dim), jnp.float32),
        scratch_shapes=[pltpu.VMEM((_N, dim), jnp.bfloat16)],
    )(x1_full, adj_local, W2, b2r)

    return x1_local, out_local


def kernel(feature, adj, W1, b1, W2, b2):
    hid = W1.shape[1]
    dim = W2.shape[1]
    b1r = b1.reshape(1, hid)
    b2r = b2.reshape(1, dim)

    devs = jax.devices()
    n_shards = 2 if len(devs) >= 2 and _N % (2 * _ROWS) == 0 else 1
    mesh = Mesh(np.asarray(devs[:n_shards]), ("x",))
    fn = jax.shard_map(
        _two_layer_local,
        mesh=mesh,
        in_specs=(P(), P("x", None), P(), P(), P(), P()),
        out_specs=(P("x", None), P("x", None)),
        check_vma=False,
    )
    return fn(feature, adj, W1, b1r, W2, b2r)


# speedup vs baseline: 2.9482x; 2.9482x over previous
import jax
import jax.numpy as jnp
from jax.experimental import pallas as pl
from jax.experimental.pallas import tpu as pltpu

_N = 10000
_ROWS = 200
_NB = _N // _ROWS
_CB = 7


def _proj_body(feat_ref, w1_ref, u_ref):
    u = jnp.dot(feat_ref[...], w1_ref[...], preferred_element_type=jnp.float32)
    u_ref[...] = u.astype(jnp.bfloat16)


def _body(u_ref, adj_ref, b1_ref, w2_ref, b2_ref,
          x1_ref, out_ref, v_ref, cache_ref, h2_ref):
    i = pl.program_id(0)

    @pl.when(i < _NB)
    def _():
        a = adj_ref[...].astype(jnp.bfloat16)

        @pl.when(i < _CB)
        def _():
            cache_ref[pl.ds(i * _ROWS, _ROWS), :] = a

        h = jnp.dot(a, u_ref[...], preferred_element_type=jnp.float32)
        x1v = jnp.maximum(h + b1_ref[...], 0.0)
        x1_ref[...] = x1v
        v = jnp.dot(x1v.astype(jnp.bfloat16), w2_ref[...].astype(jnp.bfloat16),
                    preferred_element_type=jnp.float32)
        v_ref[pl.ds(i * _ROWS, _ROWS), :] = v.astype(jnp.bfloat16)

    @pl.when(i >= _NB)
    def _():
        j = i - _NB

        @pl.when(j < _CB)
        def _():
            a = cache_ref[pl.ds(j * _ROWS, _ROWS), :]
            h2_ref[...] = jnp.dot(a, v_ref[...],
                                  preferred_element_type=jnp.float32)

        @pl.when(j >= _CB)
        def _():
            a = adj_ref[...].astype(jnp.bfloat16)
            h2_ref[...] = jnp.dot(a, v_ref[...],
                                  preferred_element_type=jnp.float32)

        h = h2_ref[...] + b2_ref[...]
        m = jnp.max(h, axis=1, keepdims=True)
        e = jnp.exp(h - m)
        s = jnp.sum(e, axis=1, keepdims=True)
        out_ref[...] = h - m - jnp.log(s)


def kernel(feature, adj, W1, b1, W2, b2):
    f_in = feature.shape[1]
    hid = W1.shape[1]
    dim = W2.shape[1]
    b1r = b1.reshape(1, hid)
    b2r = b2.reshape(1, dim)

    u = pl.pallas_call(
        _proj_body,
        in_specs=[
            pl.BlockSpec((_N, f_in), lambda: (0, 0)),
            pl.BlockSpec((f_in, hid), lambda: (0, 0)),
        ],
        out_specs=pl.BlockSpec((_N, hid), lambda: (0, 0)),
        out_shape=jax.ShapeDtypeStruct((_N, hid), jnp.bfloat16),
    )(feature, W1)

    x1, out = pl.pallas_call(
        _body,
        grid=(2 * _NB,),
        in_specs=[
            pl.BlockSpec((_N, hid), lambda i: (0, 0)),
            pl.BlockSpec((_ROWS, _N),
                         lambda i: (jnp.where(i < _NB, i,
                                              jnp.maximum(i - _NB, _CB)), 0)),
            pl.BlockSpec((1, hid), lambda i: (0, 0)),
            pl.BlockSpec((hid, dim), lambda i: (0, 0)),
            pl.BlockSpec((1, dim), lambda i: (0, 0)),
        ],
        out_specs=[
            pl.BlockSpec((_ROWS, hid),
                         lambda i: (jnp.where(i < _NB, i, _NB - 1), 0)),
            pl.BlockSpec((_ROWS, dim),
                         lambda i: (jnp.where(i < _NB, 0, i - _NB), 0)),
        ],
        out_shape=[
            jax.ShapeDtypeStruct((_N, hid), jnp.float32),
            jax.ShapeDtypeStruct((_N, dim), jnp.float32),
        ],
        scratch_shapes=[
            pltpu.VMEM((_N, dim), jnp.bfloat16),
            pltpu.VMEM((_CB * _ROWS, _N), jnp.bfloat16),
            pltpu.VMEM((_ROWS, dim), jnp.float32),
        ],
    )(u, adj, b1r, W2, b2r)
    return (x1, out)
